# baseline (speedup 1.0000x reference)
import functools

import jax
import jax.numpy as jnp
import numpy as np
from jax import lax
from jax.experimental import pallas as pl
from jax.experimental.pallas import tpu as pltpu
from jax.experimental.pallas import tpu_sc as plsc

N = 10000
E = 320000
D = 128
G = 64

_BE = 4000
_BN = 1000


def _swizzle64():
    p = np.zeros(64, np.int64)
    for g in range(2):
        b = 32 * g
        for j in range(16):
            p[b + 2 * j] = b + j
            p[b + 2 * j + 1] = b + 16 + j
    return p


_PERM64 = _swizzle64()
_PERM128 = np.concatenate([_PERM64, 64 + _PERM64])


def _edge_lin_body(ea_ref, w_ref, b_ref, o0, o1, o2, o3):
    e = jnp.dot(ea_ref[...], w_ref[...], preferred_element_type=jnp.float32)
    e = e + b_ref[...]
    for l, o in enumerate((o0, o1, o2, o3)):
        o[0] = e[:, l * 128:l * 128 + 64].astype(jnp.bfloat16)
        o[1] = e[:, l * 128 + 64:l * 128 + 128].astype(jnp.bfloat16)


def _edge_lin(edge_attr, W, b):
    grid = (E // _BE,)
    out = pl.pallas_call(
        _edge_lin_body,
        grid=grid,
        in_specs=[
            pl.BlockSpec((_BE, 16), lambda i: (i, 0)),
            pl.BlockSpec((16, 512), lambda i: (0, 0)),
            pl.BlockSpec((1, 512), lambda i: (0, 0)),
        ],
        out_specs=[pl.BlockSpec((2, _BE, 64), lambda i: (0, i, 0))] * 4,
        out_shape=[jax.ShapeDtypeStruct((2, E, 64), jnp.bfloat16)] * 4,
    )(edge_attr, W, b)
    return out


def _node_mlp_body(eps_ref, h_ref, a_ref, wa_ref, ba_ref, wb_ref, bb_ref,
                   wbp_ref, bbp_ref, o_ref, os_ref):
    a = jnp.concatenate([a_ref[0], a_ref[1]], axis=1)
    z = (1.0 + eps_ref[0]) * h_ref[...] + a
    t = jnp.dot(z, wa_ref[...], preferred_element_type=jnp.float32) + ba_ref[...]
    t = jnp.maximum(t, 0.0)
    u = jnp.dot(t, wb_ref[...], preferred_element_type=jnp.float32) + bb_ref[...]
    o_ref[...] = jnp.maximum(u, 0.0)
    up = jnp.dot(t, wbp_ref[...], preferred_element_type=jnp.float32) + bbp_ref[...]
    up = jnp.maximum(up, 0.0).astype(jnp.bfloat16)
    os_ref[0] = up[:, :64]
    os_ref[1] = up[:, 64:]


def _node_mlp(eps, h, aggr, Wa, ba, Wb, bb, Wbp, bbp):
    grid = (N // _BN,)
    return pl.pallas_call(
        _node_mlp_body,
        grid=grid,
        in_specs=[
            pl.BlockSpec(memory_space=pltpu.SMEM),
            pl.BlockSpec((_BN, 128), lambda i: (i, 0)),
            pl.BlockSpec((_NC, _BN, 64), lambda i: (0, i, 0)),
            pl.BlockSpec((128, 128), lambda i: (0, 0)),
            pl.BlockSpec((1, 128), lambda i: (0, 0)),
            pl.BlockSpec((128, 128), lambda i: (0, 0)),
            pl.BlockSpec((1, 128), lambda i: (0, 0)),
            pl.BlockSpec((128, 128), lambda i: (0, 0)),
            pl.BlockSpec((1, 128), lambda i: (0, 0)),
        ],
        out_specs=[pl.BlockSpec((_BN, 128), lambda i: (i, 0)),
                   pl.BlockSpec((2, _BN, 64), lambda i: (0, i, 0))],
        out_shape=[jax.ShapeDtypeStruct((N, 128), jnp.float32),
                   jax.ShapeDtypeStruct((2, N, 64), jnp.bfloat16)],
    )(eps.reshape(1), h, aggr, Wa, ba.reshape(1, 128), Wb, bb.reshape(1, 128),
      Wbp, bbp.reshape(1, 128))


def _readout_body(batch_s, h_ref, bv_ref, wlin_ref, blin_ref, wout_ref, bout_ref,
                  o_ref, gmax_acc, gsum_acc, gcnt_acc):
    c = pl.program_id(0)
    nb = pl.num_programs(0)

    @pl.when(c == 0)
    def _init():
        gmax_acc[...] = jnp.full((G, 128), -jnp.inf, jnp.float32)
        gsum_acc[...] = jnp.zeros((G, 128), jnp.float32)
        gcnt_acc[...] = jnp.zeros((G, 128), jnp.float32)

    rows = h_ref[...]
    bv = bv_ref[...]
    gcol = lax.broadcasted_iota(jnp.int32, (_BN, G), 1)
    onehot = (bv == gcol).astype(jnp.float32)
    dn = (((0,), (0,)), ((), ()))
    gsum_acc[...] += lax.dot_general(onehot, rows, dn,
                                     preferred_element_type=jnp.float32)
    ones = jnp.ones((_BN, 128), jnp.float32)
    gcnt_acc[...] += lax.dot_general(onehot, ones, dn,
                                     preferred_element_type=jnp.float32)

    g_lo = batch_s[c * _BN]
    g_hi = batch_s[c * _BN + _BN - 1]

    def body(g, _):
        masked = jnp.where(bv == g, rows, -jnp.inf)
        m = jnp.max(masked, axis=0, keepdims=True)
        cur = gmax_acc[pl.ds(g, 1), :]
        gmax_acc[pl.ds(g, 1), :] = jnp.maximum(cur, m)
        return 0

    lax.fori_loop(g_lo, g_hi + 1, body, 0)

    @pl.when(c == nb - 1)
    def _final():
        gmax = gmax_acc[...]
        gmean = gsum_acc[...] / jnp.maximum(gcnt_acc[...], 1.0)
        z = (jnp.dot(gmax, wlin_ref[0:128, :], preferred_element_type=jnp.float32)
             + jnp.dot(gmean, wlin_ref[128:256, :], preferred_element_type=jnp.float32)
             + blin_ref[...])
        out = jnp.dot(z, wout_ref[...], preferred_element_type=jnp.float32) + bout_ref[...]
        o_ref[...] = 1.0 / (1.0 + jnp.exp(-out))


def _readout(h, batch, Wlin, blin, Wout, bout):
    grid = (N // _BN,)
    return pl.pallas_call(
        _readout_body,
        grid=grid,
        in_specs=[
            pl.BlockSpec(memory_space=pltpu.SMEM),
            pl.BlockSpec((_BN, 128), lambda i: (i, 0)),
            pl.BlockSpec((_BN, 1), lambda i: (i, 0)),
            pl.BlockSpec((256, 256), lambda i: (0, 0)),
            pl.BlockSpec((1, 256), lambda i: (0, 0)),
            pl.BlockSpec((256, 1), lambda i: (0, 0)),
            pl.BlockSpec((1, 1), lambda i: (0, 0)),
        ],
        out_specs=pl.BlockSpec((G, 1), lambda i: (0, 0)),
        out_shape=jax.ShapeDtypeStruct((G, 1), jnp.float32),
        scratch_shapes=[
            pltpu.VMEM((G, 128), jnp.float32),
            pltpu.VMEM((G, 128), jnp.float32),
            pltpu.VMEM((G, 128), jnp.float32),
        ],
    )(batch, h, batch.reshape(N, 1), Wlin, blin.reshape(1, 256), Wout,
      bout.reshape(1, 1))


_NC = 2
_NS = 16
_EK = 80
_EPT = E // _NS
_NCH = _EPT // _EK
_NPAD = 10240
_RPT = _NPAD // _NS
_NB = 5
_FW = 64


def _mp_body(h_hbm, e_hbm, src_hbm, dst_hbm, out_hbm,
             aggr_sh, sidx, didx, ebuf, gbuf, sbuf, sem_ld, sem_g, sem_sc):
    c = lax.axis_index("c")
    s = lax.axis_index("s")
    tile_base = s * _EPT

    def zrow(i, _):
        for f in range(_FW // 16):
            sbuf[0][i, pl.ds(f * 16, 16)] = jnp.zeros((16,), jnp.float32)
        return 0
    lax.fori_loop(0, _EK, zrow, 0)
    for k in range(_RPT // _EK):
        pltpu.sync_copy(sbuf[0], aggr_sh.at[pl.ds(s * _RPT + k * _EK, _EK), :])
    plsc.subcore_barrier()

    def issue_loads(j, b):
        base = tile_base + j * _EK
        pltpu.async_copy(src_hbm.at[pl.ds(base, _EK)], sidx[b], sem_ld[b])
        pltpu.async_copy(dst_hbm.at[pl.ds(base, _EK)], didx[b], sem_ld[b])
        pltpu.async_copy(e_hbm.at[c, pl.ds(base, _EK), :], ebuf[b], sem_ld[b])

    def wait_loads(b):
        pltpu.make_async_copy(src_hbm.at[pl.ds(0, _EK)], sidx[b], sem_ld[b]).wait()
        pltpu.make_async_copy(dst_hbm.at[pl.ds(0, _EK)], didx[b], sem_ld[b]).wait()
        pltpu.make_async_copy(e_hbm.at[c, pl.ds(0, _EK), :], ebuf[b], sem_ld[b]).wait()

    def issue_gather(b):
        pltpu.async_copy(h_hbm.at[c].at[sidx[b]], gbuf[b], sem_g[b])

    def wait_gather(b):
        pltpu.make_async_copy(h_hbm.at[c].at[sidx[b]], gbuf[b], sem_g[b]).wait()

    def issue_scatter(b):
        pltpu.async_copy(sbuf[b], aggr_sh.at[didx[b]], sem_sc[b], add=True)

    def wait_scatter(b):
        pltpu.make_async_copy(sbuf[b], aggr_sh.at[didx[b]], sem_sc[b]).wait()

    def relu(b):
        def row(i, _):
            sh = jnp.full((16,), 16, jnp.uint32)
            mask = jnp.full((16,), 0xFFFF0000, jnp.uint32)
            for f in range(_FW // 32):
                ue = ebuf[b][i, pl.ds(f * 16, 16)]
                ug = gbuf[b][i, pl.ds(f * 16, 16)]
                bc = lambda x: jax.lax.bitcast_convert_type(x, jnp.float32)
                elo = bc(ue << sh)
                ehi = bc(ue & mask)
                glo = bc(ug << sh)
                ghi = bc(ug & mask)
                sbuf[b][i, pl.ds(f * 32, 16)] = jnp.maximum(elo + glo, 0.0)
                sbuf[b][i, pl.ds(f * 32 + 16, 16)] = jnp.maximum(ehi + ghi, 0.0)
            return 0
        lax.fori_loop(0, _EK, row, 0)

    for b in range(3):
        issue_loads(b, b)
    wait_loads(0)
    issue_gather(0)

    def epoch(t, _):
        for b in range(_NB):
            j = t * _NB + b
            bn = (b + 1) % _NB
            bl = (b + 3) % _NB

            @pl.when(j + 1 < _NCH)
            def _():
                wait_loads(bn)
                issue_gather(bn)

            wait_gather(b)
            relu(b)
            issue_scatter(b)

            @pl.when(j + 3 < _NCH)
            def _():
                @pl.when(j >= 2)
                def _():
                    wait_scatter(bl)
                issue_loads(j + 3, bl)
        return 0

    lax.fori_loop(0, _NCH // _NB, epoch, 0)
    for b in range(_NB):
        wait_scatter(b)
    plsc.subcore_barrier()

    pltpu.sync_copy(aggr_sh.at[pl.ds(s * _RPT, _RPT), :],
                    out_hbm.at[c, pl.ds(s * _RPT, _RPT), :])


@functools.lru_cache(maxsize=None)
def _build_message_pass_sc():
    return pl.kernel(
        _mp_body,
        out_type=jax.ShapeDtypeStruct((_NC, _NPAD, _FW), jnp.float32),
        mesh=plsc.VectorSubcoreMesh(core_axis_name="c", subcore_axis_name="s",
                                    num_cores=_NC, num_subcores=_NS),
        compiler_params=pltpu.CompilerParams(use_tc_tiling_on_sc=False),
        scratch_types=[
            pltpu.VMEM_SHARED((_NPAD, _FW), jnp.float32),
            [pltpu.VMEM((_EK,), jnp.int32)] * _NB,
            [pltpu.VMEM((_EK,), jnp.int32)] * _NB,
            [pltpu.VMEM((_EK, _FW // 2), jnp.uint32)] * _NB,
            [pltpu.VMEM((_EK, _FW // 2), jnp.uint32)] * _NB,
            [pltpu.VMEM((_EK, _FW), jnp.float32)] * _NB,
            [pltpu.SemaphoreType.DMA] * _NB,
            [pltpu.SemaphoreType.DMA] * _NB,
            [pltpu.SemaphoreType.DMA] * _NB,
        ],
    )


def _pack_u32(x):
    return jax.lax.bitcast_convert_type(
        x.reshape(*x.shape[:-1], x.shape[-1] // 2, 2), jnp.uint32)


def _message_pass(hs, es, src, dst):
    out = _build_message_pass_sc()(_pack_u32(hs), _pack_u32(es), src, dst)
    return out[:, :N, :]


def kernel(x, edge_attr, edge_index, batch,
           eps1, We1, be1, W1a, b1a, W1b, b1b,
           eps_l, We_l, be_l, Wa_l, ba_l, Wb_l, bb_l,
           Wlin, blin, Wout, bout):
    src = edge_index[0]
    dst = edge_index[1]

    W_all = jnp.concatenate([We1, We_l[0], We_l[1], We_l[2]], axis=1)
    b_all = jnp.concatenate([be1, be_l[0], be_l[1], be_l[2]])
    perm512 = np.concatenate([l * 128 + _PERM128 for l in range(4)])
    e_list = _edge_lin(edge_attr, W_all[:, perm512], b_all[perm512].reshape(1, 512))

    h = x
    xp = x[:, _PERM128].astype(jnp.bfloat16)
    hs = jnp.stack([xp[:, :64], xp[:, 64:]])
    eps_all = [eps1, eps_l[0], eps_l[1], eps_l[2]]
    Wa_all = [W1a, Wa_l[0], Wa_l[1], Wa_l[2]]
    ba_all = [b1a, ba_l[0], ba_l[1], ba_l[2]]
    Wb_all = [W1b, Wb_l[0], Wb_l[1], Wb_l[2]]
    bb_all = [b1b, bb_l[0], bb_l[1], bb_l[2]]
    for i in range(4):
        aggr = _message_pass(hs, e_list[i], src, dst)
        h, hs = _node_mlp(eps_all[i], h, aggr, Wa_all[i], ba_all[i],
                          Wb_all[i], bb_all[i],
                          Wb_all[i][:, _PERM128], bb_all[i][_PERM128])

    return _readout(h, batch, Wlin, blin, Wout, bout)


# speedup vs baseline: 2.2729x; 2.2729x over previous
import functools

import jax
import jax.numpy as jnp
import numpy as np
from jax import lax
from jax.experimental import pallas as pl
from jax.experimental.pallas import tpu as pltpu
from jax.experimental.pallas import tpu_sc as plsc

N = 10000
E = 320000
D = 128
G = 64

_BE = 4000
_BN = 1000


def _swizzle64():
    p = np.zeros(64, np.int64)
    for g in range(2):
        b = 32 * g
        for j in range(16):
            p[b + 2 * j] = b + j
            p[b + 2 * j + 1] = b + 16 + j
    return p


_PERM64 = _swizzle64()
_PERM128 = np.concatenate([_PERM64, 64 + _PERM64])


def _edge_lin_body(ea_ref, w_ref, b_ref, o0, o1, o2, o3):
    e = jnp.dot(ea_ref[...], w_ref[...], preferred_element_type=jnp.float32)
    e = e + b_ref[...]
    for l, o in enumerate((o0, o1, o2, o3)):
        o[0] = e[:, l * 128:l * 128 + 64]
        o[1] = e[:, l * 128 + 64:l * 128 + 128]


def _edge_lin(edge_attr, W, b):
    grid = (E // _BE,)
    out = pl.pallas_call(
        _edge_lin_body,
        grid=grid,
        in_specs=[
            pl.BlockSpec((_BE, 16), lambda i: (i, 0)),
            pl.BlockSpec((16, 512), lambda i: (0, 0)),
            pl.BlockSpec((1, 512), lambda i: (0, 0)),
        ],
        out_specs=[pl.BlockSpec((2, _BE, 64), lambda i: (0, i, 0))] * 4,
        out_shape=[jax.ShapeDtypeStruct((2, E, 64), jnp.float32)] * 4,
    )(edge_attr, W, b)
    return out


def _node_mlp_body(eps_ref, h_ref, a_ref, wa_ref, ba_ref, wb_ref, bb_ref,
                   o_ref, os_ref):
    a = jnp.concatenate([a_ref[0], a_ref[1]], axis=1)
    z = (1.0 + eps_ref[0]) * h_ref[...] + a
    t = jnp.dot(z, wa_ref[...], preferred_element_type=jnp.float32) + ba_ref[...]
    t = jnp.maximum(t, 0.0)
    u = jnp.dot(t, wb_ref[...], preferred_element_type=jnp.float32) + bb_ref[...]
    u = jnp.maximum(u, 0.0)
    o_ref[...] = u
    os_ref[0] = u[:, :64]
    os_ref[1] = u[:, 64:]


def _node_mlp(eps, h, aggr, Wa, ba, Wb, bb):
    grid = (N // _BN,)
    return pl.pallas_call(
        _node_mlp_body,
        grid=grid,
        in_specs=[
            pl.BlockSpec(memory_space=pltpu.SMEM),
            pl.BlockSpec((_BN, 128), lambda i: (i, 0)),
            pl.BlockSpec((_NC, _BN, 64), lambda i: (0, i, 0)),
            pl.BlockSpec((128, 128), lambda i: (0, 0)),
            pl.BlockSpec((1, 128), lambda i: (0, 0)),
            pl.BlockSpec((128, 128), lambda i: (0, 0)),
            pl.BlockSpec((1, 128), lambda i: (0, 0)),
        ],
        out_specs=[pl.BlockSpec((_BN, 128), lambda i: (i, 0)),
                   pl.BlockSpec((2, _BN, 64), lambda i: (0, i, 0))],
        out_shape=[jax.ShapeDtypeStruct((N, 128), jnp.float32),
                   jax.ShapeDtypeStruct((2, N, 64), jnp.float32)],
    )(eps.reshape(1), h, aggr, Wa, ba.reshape(1, 128), Wb, bb.reshape(1, 128))


def _readout_body(batch_s, h_ref, bv_ref, wlin_ref, blin_ref, wout_ref, bout_ref,
                  o_ref, gmax_acc, gsum_acc, gcnt_acc):
    c = pl.program_id(0)
    nb = pl.num_programs(0)

    @pl.when(c == 0)
    def _init():
        gmax_acc[...] = jnp.full((G, 128), -jnp.inf, jnp.float32)
        gsum_acc[...] = jnp.zeros((G, 128), jnp.float32)
        gcnt_acc[...] = jnp.zeros((G, 128), jnp.float32)

    rows = h_ref[...]
    bv = bv_ref[...]
    gcol = lax.broadcasted_iota(jnp.int32, (_BN, G), 1)
    onehot = (bv == gcol).astype(jnp.float32)
    dn = (((0,), (0,)), ((), ()))
    gsum_acc[...] += lax.dot_general(onehot, rows, dn,
                                     preferred_element_type=jnp.float32)
    ones = jnp.ones((_BN, 128), jnp.float32)
    gcnt_acc[...] += lax.dot_general(onehot, ones, dn,
                                     preferred_element_type=jnp.float32)

    g_lo = batch_s[c * _BN]
    g_hi = batch_s[c * _BN + _BN - 1]

    def body(g, _):
        masked = jnp.where(bv == g, rows, -jnp.inf)
        m = jnp.max(masked, axis=0, keepdims=True)
        cur = gmax_acc[pl.ds(g, 1), :]
        gmax_acc[pl.ds(g, 1), :] = jnp.maximum(cur, m)
        return 0

    lax.fori_loop(g_lo, g_hi + 1, body, 0)

    @pl.when(c == nb - 1)
    def _final():
        gmax = gmax_acc[...]
        gmean = gsum_acc[...] / jnp.maximum(gcnt_acc[...], 1.0)
        z = (jnp.dot(gmax, wlin_ref[0:128, :], preferred_element_type=jnp.float32)
             + jnp.dot(gmean, wlin_ref[128:256, :], preferred_element_type=jnp.float32)
             + blin_ref[...])
        out = jnp.dot(z, wout_ref[...], preferred_element_type=jnp.float32) + bout_ref[...]
        o_ref[...] = 1.0 / (1.0 + jnp.exp(-out))


def _readout(h, batch, Wlin, blin, Wout, bout):
    grid = (N // _BN,)
    return pl.pallas_call(
        _readout_body,
        grid=grid,
        in_specs=[
            pl.BlockSpec(memory_space=pltpu.SMEM),
            pl.BlockSpec((_BN, 128), lambda i: (i, 0)),
            pl.BlockSpec((_BN, 1), lambda i: (i, 0)),
            pl.BlockSpec((256, 256), lambda i: (0, 0)),
            pl.BlockSpec((1, 256), lambda i: (0, 0)),
            pl.BlockSpec((256, 1), lambda i: (0, 0)),
            pl.BlockSpec((1, 1), lambda i: (0, 0)),
        ],
        out_specs=pl.BlockSpec((G, 1), lambda i: (0, 0)),
        out_shape=jax.ShapeDtypeStruct((G, 1), jnp.float32),
        scratch_shapes=[
            pltpu.VMEM((G, 128), jnp.float32),
            pltpu.VMEM((G, 128), jnp.float32),
            pltpu.VMEM((G, 128), jnp.float32),
        ],
    )(batch, h, batch.reshape(N, 1), Wlin, blin.reshape(1, 256), Wout,
      bout.reshape(1, 1))


_NC = 2
_NS = 16
_EK = 80
_EPT = E // _NS
_NCH = _EPT // _EK
_NPAD = 10240
_RPT = _NPAD // _NS
_NB = 5
_RU = 4
_FW = 64


def _mp_body(h_hbm, e_hbm, src_hbm, dst_hbm, out_hbm,
             aggr_sh, sidx, didx, ebuf, sbuf, sem_ld, sem_g, sem_sc):
    c = lax.axis_index("c")
    s = lax.axis_index("s")
    tile_base = s * _EPT

    def zrow(i, _):
        for f in range(_FW // 16):
            sbuf[0][i, pl.ds(f * 16, 16)] = jnp.zeros((16,), jnp.float32)
        return 0
    lax.fori_loop(0, _EK, zrow, 0)
    for k in range(_RPT // _EK):
        pltpu.sync_copy(sbuf[0], aggr_sh.at[pl.ds(s * _RPT + k * _EK, _EK), :])
    plsc.subcore_barrier()

    def issue_loads(j, b):
        base = tile_base + j * _EK
        pltpu.async_copy(src_hbm.at[pl.ds(base, _EK)], sidx[b], sem_ld[b])
        pltpu.async_copy(dst_hbm.at[pl.ds(base, _EK)], didx[b], sem_ld[b])
        pltpu.async_copy(e_hbm.at[c, pl.ds(base, _EK), :], ebuf[b], sem_ld[b])

    def wait_loads(b):
        pltpu.make_async_copy(src_hbm.at[pl.ds(0, _EK)], sidx[b], sem_ld[b]).wait()
        pltpu.make_async_copy(dst_hbm.at[pl.ds(0, _EK)], didx[b], sem_ld[b]).wait()
        pltpu.make_async_copy(e_hbm.at[c, pl.ds(0, _EK), :], ebuf[b], sem_ld[b]).wait()

    def issue_gather(b):
        pltpu.async_copy(h_hbm.at[c].at[sidx[b]], ebuf[b], sem_g[b], add=True)

    def wait_gather(b):
        pltpu.make_async_copy(h_hbm.at[c].at[sidx[b]], ebuf[b], sem_g[b]).wait()

    def issue_scatter(b):
        pltpu.async_copy(sbuf[b], aggr_sh.at[didx[b]], sem_sc[b], add=True)

    def wait_scatter(b):
        pltpu.make_async_copy(sbuf[b], aggr_sh.at[didx[b]], sem_sc[b]).wait()

    def relu(b):
        def rows(i):
            for r in range(_RU):
                for f in range(_FW // 16):
                    v = ebuf[b][i * _RU + r, pl.ds(f * 16, 16)]
                    sbuf[b][i * _RU + r, pl.ds(f * 16, 16)] = jnp.maximum(v, 0.0)
        plsc.parallel_loop(0, _EK // _RU, 1, unroll=2)(rows)

    for b in range(3):
        issue_loads(b, b)
    wait_loads(0)
    issue_gather(0)

    def epoch(t, _):
        for b in range(_NB):
            j = t * _NB + b
            bn = (b + 1) % _NB
            bl = (b + 3) % _NB

            @pl.when(j + 1 < _NCH)
            def _():
                wait_loads(bn)
                issue_gather(bn)

            wait_gather(b)
            relu(b)
            issue_scatter(b)

            @pl.when(j + 3 < _NCH)
            def _():
                @pl.when(j >= 2)
                def _():
                    wait_scatter(bl)
                issue_loads(j + 3, bl)
        return 0

    lax.fori_loop(0, _NCH // _NB, epoch, 0)
    for b in range(_NB):
        wait_scatter(b)
    plsc.subcore_barrier()

    pltpu.sync_copy(aggr_sh.at[pl.ds(s * _RPT, _RPT), :],
                    out_hbm.at[c, pl.ds(s * _RPT, _RPT), :])


@functools.lru_cache(maxsize=None)
def _build_message_pass_sc():
    return pl.kernel(
        _mp_body,
        out_type=jax.ShapeDtypeStruct((_NC, _NPAD, _FW), jnp.float32),
        mesh=plsc.VectorSubcoreMesh(core_axis_name="c", subcore_axis_name="s",
                                    num_cores=_NC, num_subcores=_NS),
        compiler_params=pltpu.CompilerParams(use_tc_tiling_on_sc=False),
        scratch_types=[
            pltpu.VMEM_SHARED((_NPAD, _FW), jnp.float32),
            [pltpu.VMEM((_EK,), jnp.int32)] * _NB,
            [pltpu.VMEM((_EK,), jnp.int32)] * _NB,
            [pltpu.VMEM((_EK, _FW), jnp.float32)] * _NB,
            [pltpu.VMEM((_EK, _FW), jnp.float32)] * _NB,
            [pltpu.SemaphoreType.DMA] * _NB,
            [pltpu.SemaphoreType.DMA] * _NB,
            [pltpu.SemaphoreType.DMA] * _NB,
        ],
    )


def _message_pass(hs, es, src, dst):
    out = _build_message_pass_sc()(hs, es, src, dst)
    return out[:, :N, :]


def kernel(x, edge_attr, edge_index, batch,
           eps1, We1, be1, W1a, b1a, W1b, b1b,
           eps_l, We_l, be_l, Wa_l, ba_l, Wb_l, bb_l,
           Wlin, blin, Wout, bout):
    src = edge_index[0]
    dst = edge_index[1]

    W_all = jnp.concatenate([We1, We_l[0], We_l[1], We_l[2]], axis=1)
    b_all = jnp.concatenate([be1, be_l[0], be_l[1], be_l[2]]).reshape(1, 512)
    e_list = _edge_lin(edge_attr, W_all, b_all)

    h = x
    hs = jnp.stack([x[:, :64], x[:, 64:]])
    eps_all = [eps1, eps_l[0], eps_l[1], eps_l[2]]
    Wa_all = [W1a, Wa_l[0], Wa_l[1], Wa_l[2]]
    ba_all = [b1a, ba_l[0], ba_l[1], ba_l[2]]
    Wb_all = [W1b, Wb_l[0], Wb_l[1], Wb_l[2]]
    bb_all = [b1b, bb_l[0], bb_l[1], bb_l[2]]
    for i in range(4):
        aggr = _message_pass(hs, e_list[i], src, dst)
        h, hs = _node_mlp(eps_all[i], h, aggr, Wa_all[i], ba_all[i],
                          Wb_all[i], bb_all[i])

    return _readout(h, batch, Wlin, blin, Wout, bout)


# speedup vs baseline: 3.0625x; 1.3474x over previous
import functools

import jax
import jax.numpy as jnp
from jax import lax
from jax.experimental import pallas as pl
from jax.experimental.pallas import tpu as pltpu
from jax.experimental.pallas import tpu_sc as plsc

N = 10000
E = 320000
D = 128
G = 64

_BE = 4000
_BN = 1000


def _edge_lin_body(ea_ref, w_ref, b_ref, o0, o1, o2, o3):
    e = jnp.dot(ea_ref[...], w_ref[...], preferred_element_type=jnp.float32)
    e = e + b_ref[...]
    for l, o in enumerate((o0, o1, o2, o3)):
        o[...] = e[:, l * 128:(l + 1) * 128]


def _edge_lin(edge_attr, W, b):
    grid = (E // _BE,)
    return pl.pallas_call(
        _edge_lin_body,
        grid=grid,
        in_specs=[
            pl.BlockSpec((_BE, 16), lambda i: (i, 0)),
            pl.BlockSpec((16, 512), lambda i: (0, 0)),
            pl.BlockSpec((1, 512), lambda i: (0, 0)),
        ],
        out_specs=[pl.BlockSpec((_BE, 128), lambda i: (i, 0))] * 4,
        out_shape=[jax.ShapeDtypeStruct((E, 128), jnp.float32)] * 4,
    )(edge_attr, W, b)


def _node_mlp_body(eps_ref, h_ref, a_ref, wa_ref, ba_ref, wb_ref, bb_ref, o_ref):
    z = (1.0 + eps_ref[0]) * h_ref[...] + a_ref[0] + a_ref[1]
    t = jnp.dot(z, wa_ref[...], preferred_element_type=jnp.float32) + ba_ref[...]
    t = jnp.maximum(t, 0.0)
    u = jnp.dot(t, wb_ref[...], preferred_element_type=jnp.float32) + bb_ref[...]
    o_ref[...] = jnp.maximum(u, 0.0)


def _node_mlp(eps, h, aggr, Wa, ba, Wb, bb):
    grid = (N // _BN,)
    return pl.pallas_call(
        _node_mlp_body,
        grid=grid,
        in_specs=[
            pl.BlockSpec(memory_space=pltpu.SMEM),
            pl.BlockSpec((_BN, 128), lambda i: (i, 0)),
            pl.BlockSpec((2, _BN, 128), lambda i: (0, i, 0)),
            pl.BlockSpec((128, 128), lambda i: (0, 0)),
            pl.BlockSpec((1, 128), lambda i: (0, 0)),
            pl.BlockSpec((128, 128), lambda i: (0, 0)),
            pl.BlockSpec((1, 128), lambda i: (0, 0)),
        ],
        out_specs=pl.BlockSpec((_BN, 128), lambda i: (i, 0)),
        out_shape=jax.ShapeDtypeStruct((N, 128), jnp.float32),
    )(eps.reshape(1), h, aggr, Wa, ba.reshape(1, 128), Wb, bb.reshape(1, 128))


def _readout_body(batch_s, h_ref, bv_ref, wlin_ref, blin_ref, wout_ref, bout_ref,
                  o_ref, gmax_acc, gsum_acc, gcnt_acc):
    c = pl.program_id(0)
    nb = pl.num_programs(0)

    @pl.when(c == 0)
    def _init():
        gmax_acc[...] = jnp.full((G, 128), -jnp.inf, jnp.float32)
        gsum_acc[...] = jnp.zeros((G, 128), jnp.float32)
        gcnt_acc[...] = jnp.zeros((G, 128), jnp.float32)

    rows = h_ref[...]
    bv = bv_ref[...]
    gcol = lax.broadcasted_iota(jnp.int32, (_BN, G), 1)
    onehot = (bv == gcol).astype(jnp.float32)
    dn = (((0,), (0,)), ((), ()))
    gsum_acc[...] += lax.dot_general(onehot, rows, dn,
                                     preferred_element_type=jnp.float32)
    ones = jnp.ones((_BN, 128), jnp.float32)
    gcnt_acc[...] += lax.dot_general(onehot, ones, dn,
                                     preferred_element_type=jnp.float32)

    g_lo = batch_s[c * _BN]
    g_hi = batch_s[c * _BN + _BN - 1]

    def body(g, _):
        masked = jnp.where(bv == g, rows, -jnp.inf)
        m = jnp.max(masked, axis=0, keepdims=True)
        cur = gmax_acc[pl.ds(g, 1), :]
        gmax_acc[pl.ds(g, 1), :] = jnp.maximum(cur, m)
        return 0

    lax.fori_loop(g_lo, g_hi + 1, body, 0)

    @pl.when(c == nb - 1)
    def _final():
        gmax = gmax_acc[...]
        gmean = gsum_acc[...] / jnp.maximum(gcnt_acc[...], 1.0)
        z = (jnp.dot(gmax, wlin_ref[0:128, :], preferred_element_type=jnp.float32)
             + jnp.dot(gmean, wlin_ref[128:256, :], preferred_element_type=jnp.float32)
             + blin_ref[...])
        out = jnp.dot(z, wout_ref[...], preferred_element_type=jnp.float32) + bout_ref[...]
        o_ref[...] = 1.0 / (1.0 + jnp.exp(-out))


def _readout(h, batch, Wlin, blin, Wout, bout):
    grid = (N // _BN,)
    return pl.pallas_call(
        _readout_body,
        grid=grid,
        in_specs=[
            pl.BlockSpec(memory_space=pltpu.SMEM),
            pl.BlockSpec((_BN, 128), lambda i: (i, 0)),
            pl.BlockSpec((_BN, 1), lambda i: (i, 0)),
            pl.BlockSpec((256, 256), lambda i: (0, 0)),
            pl.BlockSpec((1, 256), lambda i: (0, 0)),
            pl.BlockSpec((256, 1), lambda i: (0, 0)),
            pl.BlockSpec((1, 1), lambda i: (0, 0)),
        ],
        out_specs=pl.BlockSpec((G, 1), lambda i: (0, 0)),
        out_shape=jax.ShapeDtypeStruct((G, 1), jnp.float32),
        scratch_shapes=[
            pltpu.VMEM((G, 128), jnp.float32),
            pltpu.VMEM((G, 128), jnp.float32),
            pltpu.VMEM((G, 128), jnp.float32),
        ],
    )(batch, h, batch.reshape(N, 1), Wlin, blin.reshape(1, 256), Wout,
      bout.reshape(1, 1))


_NC = 2
_NS = 16
_EK = 40
_EPT = E // (_NC * _NS)
_NCH = _EPT // _EK
_NPAD = 10240
_RPT = _NPAD // _NS
_NBE = 5
_NBS = 2
_UP = 10
_RU = 4


def _mp_body(h_hbm, e_hbm, src_hbm, dst_hbm, out_hbm,
             aggr_sh, sidx, didx, ebuf, sbuf, sem_ld, sem_g, sem_sc):
    c = lax.axis_index("c")
    s = lax.axis_index("s")
    tile_base = c * (E // _NC) + s * _EPT

    def zrow(i, _):
        for f in range(8):
            sbuf[0][i, pl.ds(f * 16, 16)] = jnp.zeros((16,), jnp.float32)
        return 0
    lax.fori_loop(0, _EK, zrow, 0)
    for k in range(_RPT // _EK):
        pltpu.sync_copy(sbuf[0], aggr_sh.at[pl.ds(s * _RPT + k * _EK, _EK), :])
    plsc.subcore_barrier()

    def issue_loads(j, b):
        base = tile_base + j * _EK
        pltpu.async_copy(src_hbm.at[pl.ds(base, _EK)], sidx[b], sem_ld[b])
        pltpu.async_copy(dst_hbm.at[pl.ds(base, _EK)], didx[b], sem_ld[b])
        pltpu.async_copy(e_hbm.at[pl.ds(base, _EK), :], ebuf[b], sem_ld[b])

    def wait_loads(b):
        pltpu.make_async_copy(src_hbm.at[pl.ds(0, _EK)], sidx[b], sem_ld[b]).wait()
        pltpu.make_async_copy(dst_hbm.at[pl.ds(0, _EK)], didx[b], sem_ld[b]).wait()
        pltpu.make_async_copy(e_hbm.at[pl.ds(0, _EK), :], ebuf[b], sem_ld[b]).wait()

    def issue_gather(b):
        pltpu.async_copy(h_hbm.at[sidx[b]], ebuf[b], sem_g[b], add=True)

    def wait_gather(b):
        pltpu.make_async_copy(h_hbm.at[sidx[b]], ebuf[b], sem_g[b]).wait()

    def issue_scatter(be, bs):
        pltpu.async_copy(sbuf[bs], aggr_sh.at[didx[be]], sem_sc[bs], add=True)

    def wait_scatter(be, bs):
        pltpu.make_async_copy(sbuf[bs], aggr_sh.at[didx[be]], sem_sc[bs]).wait()

    def relu(be, bs):
        def rows(i):
            for r in range(_RU):
                for f in range(8):
                    v = ebuf[be][i * _RU + r, pl.ds(f * 16, 16)]
                    sbuf[bs][i * _RU + r, pl.ds(f * 16, 16)] = jnp.maximum(v, 0.0)
        plsc.parallel_loop(0, _EK // _RU, 1, unroll=2)(rows)

    for b in range(3):
        issue_loads(b, b)
    wait_loads(0)
    issue_gather(0)

    def epoch(t, _):
        for u in range(_UP):
            j = t * _UP + u
            be = u % _NBE
            bs = u % _NBS

            @pl.when(j + 1 < _NCH)
            def _():
                wait_loads((u + 1) % _NBE)
                issue_gather((u + 1) % _NBE)

            wait_gather(be)

            @pl.when(j >= _NBS)
            def _():
                wait_scatter((u - _NBS) % _NBE, bs)

            relu(be, bs)
            issue_scatter(be, bs)

            @pl.when(j + 3 < _NCH)
            def _():
                issue_loads(j + 3, (u + 3) % _NBE)
        return 0

    lax.fori_loop(0, _NCH // _UP, epoch, 0)
    for u in range(_NCH - _NBS, _NCH):
        wait_scatter(u % _NBE, u % _NBS)
    plsc.subcore_barrier()

    pltpu.sync_copy(aggr_sh.at[pl.ds(s * _RPT, _RPT), :],
                    out_hbm.at[c, pl.ds(s * _RPT, _RPT), :])


@functools.lru_cache(maxsize=None)
def _build_message_pass_sc():
    return pl.kernel(
        _mp_body,
        out_type=jax.ShapeDtypeStruct((_NC, _NPAD, 128), jnp.float32),
        mesh=plsc.VectorSubcoreMesh(core_axis_name="c", subcore_axis_name="s",
                                    num_cores=_NC, num_subcores=_NS),
        scratch_types=[
            pltpu.VMEM_SHARED((_NPAD, 128), jnp.float32),
            [pltpu.VMEM((_EK,), jnp.int32)] * _NBE,
            [pltpu.VMEM((_EK,), jnp.int32)] * _NBE,
            [pltpu.VMEM((_EK, 128), jnp.float32)] * _NBE,
            [pltpu.VMEM((_EK, 128), jnp.float32)] * _NBS,
            [pltpu.SemaphoreType.DMA] * _NBE,
            [pltpu.SemaphoreType.DMA] * _NBE,
            [pltpu.SemaphoreType.DMA] * _NBS,
        ],
    )


def _message_pass(h, e, src, dst):
    return _build_message_pass_sc()(h, e, src, dst)


def kernel(x, edge_attr, edge_index, batch,
           eps1, We1, be1, W1a, b1a, W1b, b1b,
           eps_l, We_l, be_l, Wa_l, ba_l, Wb_l, bb_l,
           Wlin, blin, Wout, bout):
    src = edge_index[0]
    dst = edge_index[1]

    W_all = jnp.concatenate([We1, We_l[0], We_l[1], We_l[2]], axis=1)
    b_all = jnp.concatenate([be1, be_l[0], be_l[1], be_l[2]]).reshape(1, 512)
    e_list = _edge_lin(edge_attr, W_all, b_all)

    h = x
    eps_all = [eps1, eps_l[0], eps_l[1], eps_l[2]]
    Wa_all = [W1a, Wa_l[0], Wa_l[1], Wa_l[2]]
    ba_all = [b1a, ba_l[0], ba_l[1], ba_l[2]]
    Wb_all = [W1b, Wb_l[0], Wb_l[1], Wb_l[2]]
    bb_all = [b1b, bb_l[0], bb_l[1], bb_l[2]]
    for i in range(4):
        aggr = _message_pass(h, e_list[i], src, dst)
        h = _node_mlp(eps_all[i], h, aggr, Wa_all[i], ba_all[i],
                      Wb_all[i], bb_all[i])

    return _readout(h, batch, Wlin, blin, Wout, bout)
